# baseline (speedup 1.0000x reference)
import functools

import jax
import jax.numpy as jnp
from jax import lax
from jax.experimental import pallas as pl
from jax.experimental.pallas import tpu as pltpu
from jax.experimental.pallas import tpu_sc as plsc

NC, NS = 2, 16
NW = NC * NS


def _sc_gather(table, idx, B, D):
    b_per_w = B // NW
    mesh = plsc.VectorSubcoreMesh(
        core_axis_name="c", subcore_axis_name="s",
        num_cores=NC, num_subcores=NS)

    @functools.partial(
        pl.kernel, mesh=mesh,
        out_type=jax.ShapeDtypeStruct((B, D), jnp.float32),
        scratch_types=[
            pltpu.VMEM((b_per_w,), jnp.int32),
            pltpu.VMEM((b_per_w, D), jnp.float32),
            pltpu.SemaphoreType.DMA,
        ],
    )
    def k(table_hbm, idx_hbm, out_hbm, idx_v, rows_v, sem):
        wid = lax.axis_index("s") * NC + lax.axis_index("c")
        base = wid * b_per_w
        pltpu.sync_copy(idx_hbm.at[pl.ds(base, b_per_w)], idx_v)
        pltpu.async_copy(table_hbm.at[idx_v], rows_v, sem).wait()
        pltpu.sync_copy(rows_v, out_hbm.at[pl.ds(base, b_per_w)])

    return k(table, idx)


def _tc_body(emb2_ref, par_ref, s0_ref, s1_ref, s2_ref, w_ref, b_ref, out_ref):
    w = w_ref[...]
    we = w[:, 0:64]
    part = (s0_ref[...] * w[:, 64:128]
            + s1_ref[...] * w[:, 128:192]
            + s2_ref[...] * w[:, 192:256])
    emb2 = emb2_ref[...]
    lo = jnp.sum(emb2[:, 0:64] * we, axis=1, keepdims=True)
    hi = jnp.sum(emb2[:, 64:128] * we, axis=1, keepdims=True)
    emb_dot = jnp.where(par_ref[...] > 0, hi, lo)
    acc = jnp.sum(part, axis=1, keepdims=True) + emb_dot + b_ref[0, 0]
    out_ref[...] = jax.nn.sigmoid(acc)


def kernel(node_idx, signal_0, signal_1, signal_2, node_embed, fc_w, fc_b):
    B, D = signal_0.shape
    n_nodes = node_embed.shape[0]
    idx = node_idx.astype(jnp.int32)
    table2 = node_embed.reshape(n_nodes // 2, 2 * D)
    emb2 = _sc_gather(table2, idx >> 1, B, 2 * D)
    par = (idx & 1).reshape(B, 1)

    BLK = 2048
    grid = (B // BLK,)
    sig_spec = pl.BlockSpec((BLK, D), lambda i: (i, 0))
    p = pl.pallas_call(
        _tc_body,
        grid=grid,
        in_specs=[pl.BlockSpec((BLK, 2 * D), lambda i: (i, 0)),
                  pl.BlockSpec((BLK, 1), lambda i: (i, 0)),
                  sig_spec, sig_spec, sig_spec,
                  pl.BlockSpec((1, 4 * D), lambda i: (0, 0)),
                  pl.BlockSpec((1, 1), lambda i: (0, 0))],
        out_specs=pl.BlockSpec((BLK, 1), lambda i: (i, 0)),
        out_shape=jax.ShapeDtypeStruct((B, 1), jnp.float32),
    )(emb2, par, signal_0, signal_1, signal_2, fc_w, fc_b.reshape(1, 1))

    return (p, jnp.float32(0.0))


# speedup vs baseline: 1.6468x; 1.6468x over previous
import functools

import jax
import jax.numpy as jnp
from jax import lax
from jax.experimental import pallas as pl
from jax.experimental.pallas import tpu as pltpu
from jax.experimental.pallas import tpu_sc as plsc

NC, NS = 2, 16
NW = NC * NS
CHUNK = 16


def _sc_gather(table, idx, B, D):
    b_per_w = B // NW
    mesh = plsc.VectorSubcoreMesh(
        core_axis_name="c", subcore_axis_name="s",
        num_cores=NC, num_subcores=NS)

    @functools.partial(
        pl.kernel, mesh=mesh,
        out_type=jax.ShapeDtypeStruct((B, D), jnp.float32),
        scratch_types=[
            pltpu.VMEM((b_per_w,), jnp.int32),
            pltpu.VMEM((b_per_w, D), jnp.float32),
            pltpu.SemaphoreType.DMA,
        ],
    )
    def k(table_hbm, idx_hbm, out_hbm, idx_v, rows_v, sem):
        wid = lax.axis_index("s") * NC + lax.axis_index("c")
        base = wid * b_per_w

        pltpu.sync_copy(idx_hbm.at[pl.ds(base, b_per_w)], idx_v)

        def wave(c, _):
            vec = idx_v[pl.ds(c * CHUNK, CHUNK)]
            descs = []
            for j in range(CHUNK):
                i = c * CHUNK + j
                descs.append(pltpu.async_copy(
                    table_hbm.at[pl.ds(vec[j], 1), :],
                    rows_v.at[pl.ds(i, 1), :], sem))
            for d in descs:
                d.wait()
            return 0

        lax.fori_loop(0, b_per_w // CHUNK, wave, 0)
        pltpu.sync_copy(rows_v, out_hbm.at[pl.ds(base, b_per_w)])

    return k(table, idx)


def _tc_body(emb_ref, s0_ref, s1_ref, s2_ref, w_ref, b_ref, out_ref):
    w = w_ref[...]
    part = (emb_ref[...] * w[:, 0:64]
            + s0_ref[...] * w[:, 64:128]
            + s1_ref[...] * w[:, 128:192]
            + s2_ref[...] * w[:, 192:256])
    acc = jnp.sum(part, axis=1, keepdims=True) + b_ref[0, 0]
    out_ref[...] = jax.nn.sigmoid(acc)


def kernel(node_idx, signal_0, signal_1, signal_2, node_embed, fc_w, fc_b):
    B, D = signal_0.shape
    emb = _sc_gather(node_embed, node_idx.astype(jnp.int32), B, D)

    BLK = 2048
    grid = (B // BLK,)
    sig_spec = pl.BlockSpec((BLK, D), lambda i: (i, 0))
    p = pl.pallas_call(
        _tc_body,
        grid=grid,
        in_specs=[sig_spec, sig_spec, sig_spec, sig_spec,
                  pl.BlockSpec((1, 4 * D), lambda i: (0, 0)),
                  pl.BlockSpec((1, 1), lambda i: (0, 0))],
        out_specs=pl.BlockSpec((BLK, 1), lambda i: (i, 0)),
        out_shape=jax.ShapeDtypeStruct((B, 1), jnp.float32),
    )(emb, signal_0, signal_1, signal_2, fc_w, fc_b.reshape(1, 1))

    return (p, jnp.float32(0.0))


# speedup vs baseline: 6.5093x; 3.9527x over previous
import functools

import jax
import jax.numpy as jnp
from jax import lax
from jax.experimental import pallas as pl
from jax.experimental.pallas import tpu as pltpu
from jax.experimental.pallas import tpu_sc as plsc

NC, NS = 2, 16
NW = NC * NS


def _dot_body(w_ref, tT_ref, d_ref):
    w_e = w_ref[:, 0:64]
    d_ref[...] = jnp.dot(w_e, tT_ref[...],
                         preferred_element_type=jnp.float32)[0]


def _table_dot(tableT, fc_w, n_nodes):
    W = 32768
    grid = (pl.cdiv(n_nodes, W),)
    return pl.pallas_call(
        _dot_body,
        grid=grid,
        in_specs=[pl.BlockSpec((1, 256), lambda i: (0, 0)),
                  pl.BlockSpec((64, W), lambda i: (0, i))],
        out_specs=pl.BlockSpec((W,), lambda i: (i,)),
        out_shape=jax.ShapeDtypeStruct((n_nodes,), jnp.float32),
    )(fc_w, tableT)


def _sc_gather_1d(d, idx, B):
    b_per_w = B // NW
    rows = b_per_w // 128
    mesh = plsc.VectorSubcoreMesh(
        core_axis_name="c", subcore_axis_name="s",
        num_cores=NC, num_subcores=NS)

    @functools.partial(
        pl.kernel, mesh=mesh,
        out_type=jax.ShapeDtypeStruct((B,), jnp.float32),
        scratch_types=[
            pltpu.VMEM((rows, 128), jnp.int32),
            pltpu.VMEM((b_per_w,), jnp.float32),
            pltpu.SemaphoreType.DMA,
        ],
    )
    def k(d_hbm, idx_hbm, out_hbm, idx_m, g_v, sem):
        wid = lax.axis_index("s") * NC + lax.axis_index("c")
        base = wid * b_per_w
        for r in range(rows):
            pltpu.sync_copy(idx_hbm.at[pl.ds(base + r * 128, 128)],
                            idx_m.at[r])
        descs = [
            pltpu.async_copy(d_hbm.at[idx_m.at[r]],
                             g_v.at[pl.ds(r * 128, 128)], sem)
            for r in range(rows)
        ]
        for dsc in descs:
            dsc.wait()
        pltpu.sync_copy(g_v, out_hbm.at[pl.ds(base, b_per_w)])

    return k(d, idx)


def _sig_body(s0_ref, s1_ref, s2_ref, w_ref, b_ref, out_ref):
    w = w_ref[...]
    acc = (jnp.dot(w[:, 64:128], s0_ref[...],
                   preferred_element_type=jnp.float32)
           + jnp.dot(w[:, 128:192], s1_ref[...],
                     preferred_element_type=jnp.float32)
           + jnp.dot(w[:, 192:256], s2_ref[...],
                     preferred_element_type=jnp.float32))
    out_ref[...] = acc + b_ref[0, 0]


def _fin_body(emb_ref, sig_ref, out_ref):
    out_ref[...] = jax.nn.sigmoid(emb_ref[...] + sig_ref[...])


def kernel(node_idx, signal_0, signal_1, signal_2, node_embed, fc_w, fc_b):
    B, D = signal_0.shape
    n_nodes = node_embed.shape[0]

    d = _table_dot(node_embed.T, fc_w, n_nodes)
    emb_dot = _sc_gather_1d(d, node_idx.astype(jnp.int32), B)

    BLK = 4096
    grid = (B // BLK,)
    sigT_spec = pl.BlockSpec((D, BLK), lambda i: (0, i))
    sig_dot = pl.pallas_call(
        _sig_body,
        grid=grid,
        in_specs=[sigT_spec, sigT_spec, sigT_spec,
                  pl.BlockSpec((1, 4 * D), lambda i: (0, 0)),
                  pl.BlockSpec((1, 1), lambda i: (0, 0))],
        out_specs=pl.BlockSpec((1, BLK), lambda i: (0, i)),
        out_shape=jax.ShapeDtypeStruct((1, B), jnp.float32),
    )(signal_0.T, signal_1.T, signal_2.T, fc_w, fc_b.reshape(1, 1))

    p = pl.pallas_call(
        _fin_body,
        grid=(1,),
        in_specs=[pl.BlockSpec((1, B), lambda i: (0, 0)),
                  pl.BlockSpec((1, B), lambda i: (0, 0))],
        out_specs=pl.BlockSpec((1, B), lambda i: (0, 0)),
        out_shape=jax.ShapeDtypeStruct((1, B), jnp.float32),
    )(emb_dot.reshape(1, B), sig_dot)

    return (p.reshape(B, 1), jnp.float32(0.0))
